# Initial kernel scaffold; baseline (speedup 1.0000x reference)
import jax
import jax.numpy as jnp
from jax.experimental import pallas as pl


def kernel(x, edge_index, W1, b1, W2, b2, Wfc, bfc):
    raise NotImplementedError("write your pallas kernel here")


# speedup vs baseline: 74.6404x; 74.6404x over previous
import functools
import jax
import jax.numpy as jnp
from jax import lax
from jax.experimental import pallas as pl
from jax.experimental.pallas import tpu as pltpu
from jax.experimental.pallas import tpu_sc as plsc

N_NODES = 100000
N_EDGES = 6400000
NPAD = 100352
NC, NS, L = 2, 16, 16
NW = NC * NS
KI = 8
RW = 1568
EPAD = RW * NW * 128
F = 16


def _mesh():
    return plsc.VectorSubcoreMesh(
        core_axis_name="c", subcore_axis_name="s", num_cores=NC, num_subcores=NS
    )


def _worker_id():
    return lax.axis_index("s") * NC + lax.axis_index("c")


def _deg_body(cidx_hbm, zeros_hbm, out_hbm, cidx_v, ones_v, acc_sh):
    cid = lax.axis_index("c")
    sid = lax.axis_index("s")
    w = _worker_id()
    seg = NPAD // NS
    pltpu.sync_copy(zeros_hbm.at[pl.ds(sid * seg, seg)], acc_sh.at[pl.ds(sid * seg, seg)])
    for i in range(128 // L):
        ones_v[pl.ds(i * L, L)] = jnp.ones((L,), jnp.float32)
    plsc.subcore_barrier()

    def step(t, _):
        base = w * RW + t * KI
        pltpu.sync_copy(cidx_hbm.at[pl.ds(base, KI)], cidx_v)
        for j in range(KI):
            pltpu.sync_copy(ones_v, acc_sh.at[cidx_v.at[j]], add=True)
        return _

    lax.fori_loop(0, RW // KI, step, None)
    plsc.subcore_barrier()
    pltpu.sync_copy(acc_sh.at[pl.ds(sid * seg, seg)],
                    out_hbm.at[pl.ds(cid * NPAD + sid * seg, seg)])


def _deg_pass(cidx2, zeros1):
    k = pl.kernel(
        _deg_body,
        out_type=jax.ShapeDtypeStruct((NC * NPAD,), jnp.float32),
        mesh=_mesh(),
        compiler_params=pltpu.CompilerParams(use_tc_tiling_on_sc=False),
        scratch_types=[
            pltpu.VMEM((KI, 128), jnp.int32),
            pltpu.VMEM((128,), jnp.float32),
            pltpu.VMEM_SHARED((NPAD,), jnp.float32),
        ],
    )
    return k(cidx2, zeros1)


def _prop_body(ridx_hbm, cidx_hbm, src_hbm, zeros_hbm, out_hbm,
               ridx_v, cidx_v, gbuf_v, sem, acc_sh):
    cid = lax.axis_index("c")
    sid = lax.axis_index("s")
    w = _worker_id()
    seg = NPAD // NS
    pltpu.sync_copy(zeros_hbm.at[pl.ds(sid * seg, seg)], acc_sh.at[pl.ds(sid * seg, seg)])
    plsc.subcore_barrier()

    def step(t, _):
        base = w * RW + t * KI
        pltpu.sync_copy(ridx_hbm.at[pl.ds(base, KI)], ridx_v)
        pltpu.sync_copy(cidx_hbm.at[pl.ds(base, KI)], cidx_v)
        cps = [pltpu.async_copy(src_hbm.at[ridx_v.at[j]], gbuf_v.at[j], sem)
               for j in range(KI)]
        for cp in cps:
            cp.wait()
        for j in range(KI):
            pltpu.sync_copy(gbuf_v.at[j], acc_sh.at[cidx_v.at[j]], add=True)
        return _

    lax.fori_loop(0, RW // KI, step, None)
    plsc.subcore_barrier()
    pltpu.sync_copy(acc_sh.at[pl.ds(sid * seg, seg)],
                    out_hbm.at[pl.ds(cid * NPAD + sid * seg, seg)])


def _prop_pass(ridx2, cidx2, src, zeros2):
    k = pl.kernel(
        _prop_body,
        out_type=jax.ShapeDtypeStruct((NC * NPAD, F), jnp.float32),
        mesh=_mesh(),
        compiler_params=pltpu.CompilerParams(use_tc_tiling_on_sc=False),
        scratch_types=[
            pltpu.VMEM((KI, 128), jnp.int32),
            pltpu.VMEM((KI, 128), jnp.int32),
            pltpu.VMEM((KI, 128, F), jnp.float32),
            pltpu.SemaphoreType.DMA,
            pltpu.VMEM_SHARED((NPAD, F), jnp.float32),
        ],
    )
    return k(ridx2, cidx2, src, zeros2)


def _prop1_body(ridx_hbm, cidx_hbm, src_hbm, zeros_hbm, out_hbm,
                ridx_v, cidx_v, gbuf_v, sem, acc_sh):
    cid = lax.axis_index("c")
    sid = lax.axis_index("s")
    w = _worker_id()
    seg = NPAD // NS
    pltpu.sync_copy(zeros_hbm.at[pl.ds(sid * seg, seg)], acc_sh.at[pl.ds(sid * seg, seg)])
    plsc.subcore_barrier()

    def step(t, _):
        base = w * RW + t * KI
        pltpu.sync_copy(ridx_hbm.at[pl.ds(base, KI)], ridx_v)
        pltpu.sync_copy(cidx_hbm.at[pl.ds(base, KI)], cidx_v)
        cps = [pltpu.async_copy(src_hbm.at[ridx_v.at[j]], gbuf_v.at[j], sem)
               for j in range(KI)]
        for cp in cps:
            cp.wait()
        for j in range(KI):
            pltpu.sync_copy(gbuf_v.at[j], acc_sh.at[cidx_v.at[j]], add=True)
        return _

    lax.fori_loop(0, RW // KI, step, None)
    plsc.subcore_barrier()
    pltpu.sync_copy(acc_sh.at[pl.ds(sid * seg, seg)],
                    out_hbm.at[pl.ds(cid * NPAD + sid * seg, seg)])


def _prop1_pass(ridx2, cidx2, src1, zeros1):
    k = pl.kernel(
        _prop1_body,
        out_type=jax.ShapeDtypeStruct((NC * NPAD,), jnp.float32),
        mesh=_mesh(),
        compiler_params=pltpu.CompilerParams(use_tc_tiling_on_sc=False),
        scratch_types=[
            pltpu.VMEM((KI, 128), jnp.int32),
            pltpu.VMEM((KI, 128), jnp.int32),
            pltpu.VMEM((KI, 128), jnp.float32),
            pltpu.SemaphoreType.DMA,
            pltpu.VMEM_SHARED((NPAD,), jnp.float32),
        ],
    )
    return k(ridx2, cidx2, src1, zeros1)


def _tc_dinv_body(deg_ref, dinv_ref):
    d = deg_ref[0, :] + deg_ref[1, :] + 1.0
    dinv_ref[0, :] = lax.rsqrt(d)


def _tc_dinv(deg2):
    return pl.pallas_call(
        _tc_dinv_body,
        out_shape=jax.ShapeDtypeStruct((1, NPAD), jnp.float32),
    )(deg2)


def _tc_xs_body(xT_ref, dinv_ref, xsT_ref):
    xsT_ref[...] = xT_ref[...] * dinv_ref[...]


def _tc_xs(xT, dinv1):
    return pl.pallas_call(
        _tc_xs_body,
        out_shape=jax.ShapeDtypeStruct((F, NPAD), jnp.float32),
    )(xT, dinv1)


def _tc_mid_body(p1aT_ref, p1bT_ref, xsT_ref, dinv_ref, W1T_ref, b1_ref,
                 WfcT_ref, W2T_ref, ssT_ref):
    hp = lax.Precision.HIGHEST
    pre = (p1aT_ref[...] + p1bT_ref[...] + xsT_ref[...]) * dinv_ref[...]
    h = jnp.maximum(
        jnp.dot(W1T_ref[...], pre, preferred_element_type=jnp.float32,
                precision=hp)
        + b1_ref[...], 0.0)
    w2fT = jnp.dot(WfcT_ref[...], W2T_ref[...],
                   preferred_element_type=jnp.float32, precision=hp)
    sT = jnp.dot(w2fT, h, preferred_element_type=jnp.float32,
                 precision=hp)
    ssT_ref[...] = sT * dinv_ref[...]


def _tc_mid(p1aT, p1bT, xsT, dinv1, W1T, b1c, WfcT, W2T):
    return pl.pallas_call(
        _tc_mid_body,
        out_shape=jax.ShapeDtypeStruct((1, NPAD), jnp.float32),
    )(p1aT, p1bT, xsT, dinv1, W1T, b1c, WfcT, W2T)


def _tc_out_body(p2_ref, ss_ref, dinv_ref, b2_ref, Wfc_ref, bfc_ref, out_ref):
    c0 = jnp.sum(b2_ref[0, :] * Wfc_ref[:, 0]) + bfc_ref[0]
    out_ref[0, :] = (p2_ref[0, :] + p2_ref[1, :] + ss_ref[0, :]) * dinv_ref[0, :] + c0


def _tc_out(p2v, ss1, dinv1, b2r, Wfc, bfc):
    return pl.pallas_call(
        _tc_out_body,
        out_shape=jax.ShapeDtypeStruct((1, NPAD), jnp.float32),
    )(p2v, ss1, dinv1, b2r, Wfc, bfc)


@jax.jit
def kernel(x, edge_index, W1, b1, W2, b2, Wfc, bfc):
    r = edge_index[0].astype(jnp.int32)
    c = edge_index[1].astype(jnp.int32)
    npadex = EPAD - N_EDGES
    rpad = (jnp.arange(npadex, dtype=jnp.int32) % N_NODES)
    cpad = N_NODES + (jnp.arange(npadex, dtype=jnp.int32) % (NPAD - N_NODES))
    ridx2 = jnp.concatenate([r, rpad]).reshape(EPAD // 128, 128)
    cidx2 = jnp.concatenate([c, cpad]).reshape(EPAD // 128, 128)

    zeros1 = jnp.zeros((NPAD,), jnp.float32)
    zeros2 = jnp.zeros((NPAD, F), jnp.float32)

    degp = _deg_pass(cidx2, zeros1).reshape(NC, NPAD)
    dinv1 = _tc_dinv(degp)

    xT = jnp.pad(x.T, ((0, F - x.shape[1]), (0, NPAD - N_NODES)))
    xsT = _tc_xs(xT, dinv1)
    xs = xsT.T
    p1p = _prop_pass(ridx2, cidx2, xs, zeros2)
    p1aT, p1bT = p1p[:NPAD].T, p1p[NPAD:].T

    W1T = jnp.pad(W1, ((0, F - W1.shape[0]), (0, 0))).T
    ss1 = _tc_mid(p1aT, p1bT, xsT, dinv1, W1T, b1[:, None], Wfc.T, W2.T)
    ss = ss1.reshape(NPAD)

    p2p = _prop1_pass(ridx2, cidx2, ss, zeros1).reshape(NC, NPAD)
    out1 = _tc_out(p2p, ss1, dinv1, b2[None, :], Wfc, bfc)
    return out1[0, :N_NODES, None]


# speedup vs baseline: 90.3355x; 1.2103x over previous
import jax
import jax.numpy as jnp
from jax import lax
from jax.experimental import pallas as pl
from jax.experimental.pallas import tpu as pltpu
from jax.experimental.pallas import tpu_sc as plsc

N_NODES = 100000
N_EDGES = 6400000
NPAD = 100352
NC, NS, L = 2, 16, 16
NW = NC * NS
KI = 8
KP = 4
RW = 1568
EPAD = RW * NW * 128
F = 16


def _mesh():
    return plsc.VectorSubcoreMesh(
        core_axis_name="c", subcore_axis_name="s", num_cores=NC, num_subcores=NS
    )


def _sc_params():
    return pltpu.CompilerParams(use_tc_tiling_on_sc=False)


def _make_pass_body(feat, ki):
    has_gather = feat >= 1
    rank2 = feat > 1

    def body(*refs):
        if has_gather:
            (ridx_hbm, cidx_hbm, src_hbm, zeros_hbm, out_hbm,
             ridx_v, cidx_v, gbuf_v, gs0, gs1, acc_sh) = refs
        else:
            (cidx_hbm, zeros_hbm, out_hbm, cidx_v, ones_v, acc_sh) = refs
        cid = lax.axis_index("c")
        sid = lax.axis_index("s")
        w = sid * NC + cid
        gsems = (gs0, gs1) if has_gather else (None, None)

        seg = NPAD // NS
        pltpu.sync_copy(zeros_hbm.at[pl.ds(sid * seg, seg)],
                        acc_sh.at[pl.ds(sid * seg, seg)])
        if not has_gather:
            for i in range(128 // L):
                ones_v[pl.ds(i * L, L)] = jnp.ones((L,), jnp.float32)
        plsc.subcore_barrier()

        def gbuf_at(b, j):
            return gbuf_v.at[b, j] if rank2 else gbuf_v.at[b * ki + j]

        def scat_fire(b):
            for j in range(ki):
                pltpu.sync_copy(gbuf_at(b, j) if has_gather else ones_v,
                                acc_sh.at[cidx_v.at[b * ki + j]], add=True)

        def outer(o, carry):
            base = w * RW + 2 * o * ki
            pltpu.sync_copy(cidx_hbm.at[pl.ds(base, 2 * ki)], cidx_v)
            if has_gather:
                pltpu.sync_copy(ridx_hbm.at[pl.ds(base, 2 * ki)], ridx_v)
                cps0 = [pltpu.async_copy(src_hbm.at[ridx_v.at[j]],
                                         gbuf_at(0, j), gsems[0])
                        for j in range(ki)]
                cps1 = [pltpu.async_copy(src_hbm.at[ridx_v.at[ki + j]],
                                         gbuf_at(1, j), gsems[1])
                        for j in range(ki)]
                for cp in cps0:
                    cp.wait()
            scat_fire(0)
            if has_gather:
                for cp in cps1:
                    cp.wait()
            scat_fire(1)
            return carry

        lax.fori_loop(0, RW // (2 * ki), outer, None)
        plsc.subcore_barrier()
        pltpu.sync_copy(acc_sh.at[pl.ds(sid * seg, seg)],
                        out_hbm.at[pl.ds(cid * NPAD + sid * seg, seg)])

    return body


def _deg_pass(cidx2, zeros1):
    k = pl.kernel(
        _make_pass_body(0, KI),
        out_type=jax.ShapeDtypeStruct((NC * NPAD,), jnp.float32),
        mesh=_mesh(),
        compiler_params=_sc_params(),
        scratch_types=[
            pltpu.VMEM((2 * KI, 128), jnp.int32),
            pltpu.VMEM((128,), jnp.float32),
            pltpu.VMEM_SHARED((NPAD,), jnp.float32),
        ],
    )
    return k(cidx2, zeros1)


def _prop_pass(ridx2, cidx2, src, zeros2):
    k = pl.kernel(
        _make_pass_body(F, KP),
        out_type=jax.ShapeDtypeStruct((NC * NPAD, F), jnp.float32),
        mesh=_mesh(),
        compiler_params=_sc_params(),
        scratch_types=[
            pltpu.VMEM((2 * KP, 128), jnp.int32),
            pltpu.VMEM((2 * KP, 128), jnp.int32),
            pltpu.VMEM((2, KP, 128, F), jnp.float32),
            pltpu.SemaphoreType.DMA,
            pltpu.SemaphoreType.DMA,
            pltpu.VMEM_SHARED((NPAD, F), jnp.float32),
        ],
    )
    return k(ridx2, cidx2, src, zeros2)


def _prop1_pass(ridx2, cidx2, src1, zeros1):
    k = pl.kernel(
        _make_pass_body(1, KI),
        out_type=jax.ShapeDtypeStruct((NC * NPAD,), jnp.float32),
        mesh=_mesh(),
        compiler_params=_sc_params(),
        scratch_types=[
            pltpu.VMEM((2 * KI, 128), jnp.int32),
            pltpu.VMEM((2 * KI, 128), jnp.int32),
            pltpu.VMEM((2 * KI, 128), jnp.float32),
            pltpu.SemaphoreType.DMA,
            pltpu.SemaphoreType.DMA,
            pltpu.VMEM_SHARED((NPAD,), jnp.float32),
        ],
    )
    return k(ridx2, cidx2, src1, zeros1)


def _tc_dinv_body(deg_ref, dinv_ref):
    d = deg_ref[0, :] + deg_ref[1, :] + 1.0
    dinv_ref[0, :] = lax.rsqrt(d)


def _tc_dinv(deg2):
    return pl.pallas_call(
        _tc_dinv_body,
        out_shape=jax.ShapeDtypeStruct((1, NPAD), jnp.float32),
    )(deg2)


def _tc_xs_body(xT_ref, dinv_ref, xsT_ref):
    xsT_ref[...] = xT_ref[...] * dinv_ref[...]


def _tc_xs(xT, dinv1):
    return pl.pallas_call(
        _tc_xs_body,
        out_shape=jax.ShapeDtypeStruct((F, NPAD), jnp.float32),
    )(xT, dinv1)


def _tc_mid_body(p1aT_ref, p1bT_ref, xsT_ref, dinv_ref, W1T_ref, b1_ref,
                 WfcT_ref, W2T_ref, ssT_ref):
    hp = lax.Precision.HIGHEST
    pre = (p1aT_ref[...] + p1bT_ref[...] + xsT_ref[...]) * dinv_ref[...]
    h = jnp.maximum(
        jnp.dot(W1T_ref[...], pre, preferred_element_type=jnp.float32,
                precision=hp)
        + b1_ref[...], 0.0)
    w2fT = jnp.dot(WfcT_ref[...], W2T_ref[...],
                   preferred_element_type=jnp.float32, precision=hp)
    sT = jnp.dot(w2fT, h, preferred_element_type=jnp.float32,
                 precision=hp)
    ssT_ref[...] = sT * dinv_ref[...]


def _tc_mid(p1aT, p1bT, xsT, dinv1, W1T, b1c, WfcT, W2T):
    return pl.pallas_call(
        _tc_mid_body,
        out_shape=jax.ShapeDtypeStruct((1, NPAD), jnp.float32),
    )(p1aT, p1bT, xsT, dinv1, W1T, b1c, WfcT, W2T)


def _tc_out_body(p2_ref, ss_ref, dinv_ref, b2_ref, Wfc_ref, bfc_ref, out_ref):
    c0 = jnp.sum(b2_ref[0, :] * Wfc_ref[:, 0]) + bfc_ref[0]
    out_ref[0, :] = (p2_ref[0, :] + p2_ref[1, :] + ss_ref[0, :]) * dinv_ref[0, :] + c0


def _tc_out(p2v, ss1, dinv1, b2r, Wfc, bfc):
    return pl.pallas_call(
        _tc_out_body,
        out_shape=jax.ShapeDtypeStruct((1, NPAD), jnp.float32),
    )(p2v, ss1, dinv1, b2r, Wfc, bfc)


@jax.jit
def kernel(x, edge_index, W1, b1, W2, b2, Wfc, bfc):
    r = edge_index[0].astype(jnp.int32)
    c = edge_index[1].astype(jnp.int32)
    npadex = EPAD - N_EDGES
    rpad = (jnp.arange(npadex, dtype=jnp.int32) % N_NODES)
    cpad = N_NODES + (jnp.arange(npadex, dtype=jnp.int32) % (NPAD - N_NODES))
    ridx2 = jnp.concatenate([r, rpad]).reshape(EPAD // 128, 128)
    cidx2 = jnp.concatenate([c, cpad]).reshape(EPAD // 128, 128)

    zeros1 = jnp.zeros((NPAD,), jnp.float32)
    zeros2 = jnp.zeros((NPAD, F), jnp.float32)

    degp = _deg_pass(cidx2, zeros1).reshape(NC, NPAD)
    dinv1 = _tc_dinv(degp)

    xT = jnp.pad(x.T, ((0, F - x.shape[1]), (0, NPAD - N_NODES)))
    xsT = _tc_xs(xT, dinv1)
    xs = xsT.T
    p1p = _prop_pass(ridx2, cidx2, xs, zeros2)
    p1aT, p1bT = p1p[:NPAD].T, p1p[NPAD:].T

    W1T = jnp.pad(W1, ((0, F - W1.shape[0]), (0, 0))).T
    ss1 = _tc_mid(p1aT, p1bT, xsT, dinv1, W1T, b1[:, None], Wfc.T, W2.T)
    ss = ss1.reshape(NPAD)

    p2p = _prop1_pass(ridx2, cidx2, ss, zeros1).reshape(NC, NPAD)
    out1 = _tc_out(p2p, ss1, dinv1, b2[None, :], Wfc, bfc)
    return out1[0, :N_NODES, None]


# speedup vs baseline: 106.0957x; 1.1745x over previous
import jax
import jax.numpy as jnp
from jax import lax
from jax.experimental import pallas as pl
from jax.experimental.pallas import tpu as pltpu
from jax.experimental.pallas import tpu_sc as plsc

N_NODES = 100000
N_EDGES = 6400000
NPAD = 100352
NC, NS, L = 2, 16, 16
NW = NC * NS
KI = 8
KP = 4
RW = 1568
EPAD = RW * NW * 128
F = 16


def _mesh():
    return plsc.VectorSubcoreMesh(
        core_axis_name="c", subcore_axis_name="s", num_cores=NC, num_subcores=NS
    )


def _sc_params():
    return pltpu.CompilerParams(use_tc_tiling_on_sc=False)


def _make_pass_body(feat, ki):
    has_gather = feat >= 1
    rank2 = feat > 1

    def body(*refs):
        if has_gather:
            (ridx_hbm, cidx_hbm, src_hbm, zeros_hbm, out_hbm,
             ridx_v, cidx_v, gbuf_v, gs0, gs1, ss0, ss1, acc_sh) = refs
        else:
            (cidx_hbm, zeros_hbm, out_hbm, cidx_v, ones_v, ss0, ss1,
             acc_sh) = refs
        cid = lax.axis_index("c")
        sid = lax.axis_index("s")
        w = sid * NC + cid
        gsems = (gs0, gs1) if has_gather else (None, None)
        ssems = (ss0, ss1)

        seg = NPAD // NS
        pltpu.sync_copy(zeros_hbm.at[pl.ds(sid * seg, seg)],
                        acc_sh.at[pl.ds(sid * seg, seg)])
        if not has_gather:
            for i in range(128 // L):
                ones_v[pl.ds(i * L, L)] = jnp.ones((L,), jnp.float32)
        plsc.subcore_barrier()

        def gbuf_at(b, j):
            return gbuf_v.at[b, j] if rank2 else gbuf_v.at[b * ki + j]

        def scat_fire(b):
            return [pltpu.async_copy(gbuf_at(b, j) if has_gather else ones_v,
                                     acc_sh.at[cidx_v.at[b * ki + j]],
                                     ssems[b], add=True)
                    for j in range(ki)]

        def outer(o, carry):
            base = w * RW + 2 * o * ki
            pltpu.sync_copy(cidx_hbm.at[pl.ds(base, 2 * ki)], cidx_v)
            if has_gather:
                pltpu.sync_copy(ridx_hbm.at[pl.ds(base, 2 * ki)], ridx_v)
                cps0 = [pltpu.async_copy(src_hbm.at[ridx_v.at[j]],
                                         gbuf_at(0, j), gsems[0])
                        for j in range(ki)]
                cps1 = [pltpu.async_copy(src_hbm.at[ridx_v.at[ki + j]],
                                         gbuf_at(1, j), gsems[1])
                        for j in range(ki)]
                for cp in cps0:
                    cp.wait()
            s0 = scat_fire(0)
            if has_gather:
                for cp in cps1:
                    cp.wait()
            s1 = scat_fire(1)
            for cp in s0:
                cp.wait()
            for cp in s1:
                cp.wait()
            return carry

        lax.fori_loop(0, RW // (2 * ki), outer, None)
        plsc.subcore_barrier()
        pltpu.sync_copy(acc_sh.at[pl.ds(sid * seg, seg)],
                        out_hbm.at[pl.ds(cid * NPAD + sid * seg, seg)])

    return body


def _deg_pass(cidx2, zeros1):
    k = pl.kernel(
        _make_pass_body(0, KI),
        out_type=jax.ShapeDtypeStruct((NC * NPAD,), jnp.float32),
        mesh=_mesh(),
        compiler_params=_sc_params(),
        scratch_types=[
            pltpu.VMEM((2 * KI, 128), jnp.int32),
            pltpu.VMEM((128,), jnp.float32),
            pltpu.SemaphoreType.DMA,
            pltpu.SemaphoreType.DMA,
            pltpu.VMEM_SHARED((NPAD,), jnp.float32),
        ],
    )
    return k(cidx2, zeros1)


def _prop_pass(ridx2, cidx2, src, zeros2):
    k = pl.kernel(
        _make_pass_body(F, KP),
        out_type=jax.ShapeDtypeStruct((NC * NPAD, F), jnp.float32),
        mesh=_mesh(),
        compiler_params=_sc_params(),
        scratch_types=[
            pltpu.VMEM((2 * KP, 128), jnp.int32),
            pltpu.VMEM((2 * KP, 128), jnp.int32),
            pltpu.VMEM((2, KP, 128, F), jnp.float32),
            pltpu.SemaphoreType.DMA,
            pltpu.SemaphoreType.DMA,
            pltpu.SemaphoreType.DMA,
            pltpu.SemaphoreType.DMA,
            pltpu.VMEM_SHARED((NPAD, F), jnp.float32),
        ],
    )
    return k(ridx2, cidx2, src, zeros2)


def _prop1_pass(ridx2, cidx2, src1, zeros1):
    k = pl.kernel(
        _make_pass_body(1, KI),
        out_type=jax.ShapeDtypeStruct((NC * NPAD,), jnp.float32),
        mesh=_mesh(),
        compiler_params=_sc_params(),
        scratch_types=[
            pltpu.VMEM((2 * KI, 128), jnp.int32),
            pltpu.VMEM((2 * KI, 128), jnp.int32),
            pltpu.VMEM((2 * KI, 128), jnp.float32),
            pltpu.SemaphoreType.DMA,
            pltpu.SemaphoreType.DMA,
            pltpu.SemaphoreType.DMA,
            pltpu.SemaphoreType.DMA,
            pltpu.VMEM_SHARED((NPAD,), jnp.float32),
        ],
    )
    return k(ridx2, cidx2, src1, zeros1)


def _tc_dinv_body(deg_ref, dinv_ref):
    d = deg_ref[0, :] + deg_ref[1, :] + 1.0
    dinv_ref[0, :] = lax.rsqrt(d)


def _tc_dinv(deg2):
    return pl.pallas_call(
        _tc_dinv_body,
        out_shape=jax.ShapeDtypeStruct((1, NPAD), jnp.float32),
    )(deg2)


def _tc_xs_body(xT_ref, dinv_ref, xsT_ref):
    xsT_ref[...] = xT_ref[...] * dinv_ref[...]


def _tc_xs(xT, dinv1):
    return pl.pallas_call(
        _tc_xs_body,
        out_shape=jax.ShapeDtypeStruct((F, NPAD), jnp.float32),
    )(xT, dinv1)


def _tc_mid_body(p1aT_ref, p1bT_ref, xsT_ref, dinv_ref, W1T_ref, b1_ref,
                 WfcT_ref, W2T_ref, ssT_ref):
    hp = lax.Precision.HIGHEST
    pre = (p1aT_ref[...] + p1bT_ref[...] + xsT_ref[...]) * dinv_ref[...]
    h = jnp.maximum(
        jnp.dot(W1T_ref[...], pre, preferred_element_type=jnp.float32,
                precision=hp)
        + b1_ref[...], 0.0)
    w2fT = jnp.dot(WfcT_ref[...], W2T_ref[...],
                   preferred_element_type=jnp.float32, precision=hp)
    sT = jnp.dot(w2fT, h, preferred_element_type=jnp.float32,
                 precision=hp)
    ssT_ref[...] = sT * dinv_ref[...]


def _tc_mid(p1aT, p1bT, xsT, dinv1, W1T, b1c, WfcT, W2T):
    return pl.pallas_call(
        _tc_mid_body,
        out_shape=jax.ShapeDtypeStruct((1, NPAD), jnp.float32),
    )(p1aT, p1bT, xsT, dinv1, W1T, b1c, WfcT, W2T)


def _tc_out_body(p2_ref, ss_ref, dinv_ref, b2_ref, Wfc_ref, bfc_ref, out_ref):
    c0 = jnp.sum(b2_ref[0, :] * Wfc_ref[:, 0]) + bfc_ref[0]
    out_ref[0, :] = (p2_ref[0, :] + p2_ref[1, :] + ss_ref[0, :]) * dinv_ref[0, :] + c0


def _tc_out(p2v, ss1, dinv1, b2r, Wfc, bfc):
    return pl.pallas_call(
        _tc_out_body,
        out_shape=jax.ShapeDtypeStruct((1, NPAD), jnp.float32),
    )(p2v, ss1, dinv1, b2r, Wfc, bfc)


@jax.jit
def kernel(x, edge_index, W1, b1, W2, b2, Wfc, bfc):
    r = edge_index[0].astype(jnp.int32)
    c = edge_index[1].astype(jnp.int32)
    npadex = EPAD - N_EDGES
    rpad = (jnp.arange(npadex, dtype=jnp.int32) % N_NODES)
    cpad = N_NODES + (jnp.arange(npadex, dtype=jnp.int32) % (NPAD - N_NODES))
    ridx2 = jnp.concatenate([r, rpad]).reshape(EPAD // 128, 128)
    cidx2 = jnp.concatenate([c, cpad]).reshape(EPAD // 128, 128)

    zeros1 = jnp.zeros((NPAD,), jnp.float32)
    zeros2 = jnp.zeros((NPAD, F), jnp.float32)

    degp = _deg_pass(cidx2, zeros1).reshape(NC, NPAD)
    dinv1 = _tc_dinv(degp)

    xT = jnp.pad(x.T, ((0, F - x.shape[1]), (0, NPAD - N_NODES)))
    xsT = _tc_xs(xT, dinv1)
    xs = xsT.T
    p1p = _prop_pass(ridx2, cidx2, xs, zeros2)
    p1aT, p1bT = p1p[:NPAD].T, p1p[NPAD:].T

    W1T = jnp.pad(W1, ((0, F - W1.shape[0]), (0, 0))).T
    ss1 = _tc_mid(p1aT, p1bT, xsT, dinv1, W1T, b1[:, None], Wfc.T, W2.T)
    ss = ss1.reshape(NPAD)

    p2p = _prop1_pass(ridx2, cidx2, ss, zeros1).reshape(NC, NPAD)
    out1 = _tc_out(p2p, ss1, dinv1, b2[None, :], Wfc, bfc)
    return out1[0, :N_NODES, None]


# speedup vs baseline: 130.6713x; 1.2316x over previous
import jax
import jax.numpy as jnp
from jax import lax
from jax.experimental import pallas as pl
from jax.experimental.pallas import tpu as pltpu
from jax.experimental.pallas import tpu_sc as plsc

N_NODES = 100000
N_EDGES = 6400000
NPAD = 100352
NC, NS, L = 2, 16, 16
NW = NC * NS
KI = 4
KP = 4
RW = 1568
EPAD = RW * NW * 128
F = 16


def _mesh():
    return plsc.VectorSubcoreMesh(
        core_axis_name="c", subcore_axis_name="s", num_cores=NC, num_subcores=NS
    )


def _sc_params():
    return pltpu.CompilerParams(use_tc_tiling_on_sc=False)


def _make_pass_body(feat, ki):
    has_gather = feat >= 1
    rank2 = feat > 1

    def body(*refs):
        if has_gather:
            (ridx_hbm, cidx_hbm, src_hbm, zeros_hbm, out_hbm,
             ridx_v, cidx_v, gbuf_v, ir0, ir1, ic0, ic1, gs0, gs1, ss0, ss1,
             acc_sh) = refs
        else:
            (cidx_hbm, zeros_hbm, out_hbm, cidx_v, ones_v, ic0, ic1,
             ss0, ss1, acc_sh) = refs
        cid = lax.axis_index("c")
        sid = lax.axis_index("s")
        w = sid * NC + cid
        rsems = (ir0, ir1) if has_gather else (None, None)
        csems = (ic0, ic1)
        gsems = (gs0, gs1) if has_gather else (None, None)
        ssems = (ss0, ss1)
        nsub = RW // (2 * ki)

        seg = NPAD // NS
        pltpu.sync_copy(zeros_hbm.at[pl.ds(sid * seg, seg)],
                        acc_sh.at[pl.ds(sid * seg, seg)])
        if not has_gather:
            for i in range(128 // L):
                ones_v[pl.ds(i * L, L)] = jnp.ones((L,), jnp.float32)
        plsc.subcore_barrier()

        def gbuf_at(b, j):
            return gbuf_v.at[b, j] if rank2 else gbuf_v.at[b * ki + j]

        def idx_fire(t, p):
            base = w * RW + t * 2 * ki
            if has_gather:
                pltpu.async_copy(ridx_hbm.at[pl.ds(base, 2 * ki)],
                                 ridx_v.at[p], rsems[p])
            pltpu.async_copy(cidx_hbm.at[pl.ds(base, 2 * ki)],
                             cidx_v.at[p], csems[p])

        def idx_drain(t, p):
            base = w * RW + t * 2 * ki
            if has_gather:
                pltpu.make_async_copy(ridx_hbm.at[pl.ds(base, 2 * ki)],
                                      ridx_v.at[p], rsems[p]).wait()
            pltpu.make_async_copy(cidx_hbm.at[pl.ds(base, 2 * ki)],
                                  cidx_v.at[p], csems[p]).wait()

        def scat_fire(p, b):
            return [pltpu.async_copy(gbuf_at(b, j) if has_gather else ones_v,
                                     acc_sh.at[cidx_v.at[p, b * ki + j]],
                                     ssems[b], add=True)
                    for j in range(ki)]

        def sub(t, p, last):
            idx_drain(t, p)
            if has_gather:
                cps0 = [pltpu.async_copy(src_hbm.at[ridx_v.at[p, j]],
                                         gbuf_at(0, j), gsems[0])
                        for j in range(ki)]
                cps1 = [pltpu.async_copy(src_hbm.at[ridx_v.at[p, ki + j]],
                                         gbuf_at(1, j), gsems[1])
                        for j in range(ki)]
                for cp in cps0:
                    cp.wait()
            s0 = scat_fire(p, 0)
            if has_gather:
                for cp in cps1:
                    cp.wait()
            s1 = scat_fire(p, 1)
            for cp in s0:
                cp.wait()
            for cp in s1:
                cp.wait()
            @pl.when(jnp.logical_not(last))
            def _():
                idx_fire(t + 2, p)

        idx_fire(0, 0)
        idx_fire(1, 1)

        def outer(o, carry):
            last = o >= nsub // 2 - 1
            sub(2 * o, 0, last)
            sub(2 * o + 1, 1, last)
            return carry

        lax.fori_loop(0, nsub // 2, outer, None)
        plsc.subcore_barrier()
        pltpu.sync_copy(acc_sh.at[pl.ds(sid * seg, seg)],
                        out_hbm.at[pl.ds(cid * NPAD + sid * seg, seg)])

    return body


def _deg_pass(cidx2, zeros1):
    k = pl.kernel(
        _make_pass_body(0, KI),
        out_type=jax.ShapeDtypeStruct((NC * NPAD,), jnp.float32),
        mesh=_mesh(),
        compiler_params=_sc_params(),
        scratch_types=[
            pltpu.VMEM((2, 2 * KI, 128), jnp.int32),
            pltpu.VMEM((128,), jnp.float32),
            pltpu.SemaphoreType.DMA,
            pltpu.SemaphoreType.DMA,
            pltpu.SemaphoreType.DMA,
            pltpu.SemaphoreType.DMA,
            pltpu.VMEM_SHARED((NPAD,), jnp.float32),
        ],
    )
    return k(cidx2, zeros1)


def _prop_pass(ridx2, cidx2, src, zeros2):
    k = pl.kernel(
        _make_pass_body(F, KP),
        out_type=jax.ShapeDtypeStruct((NC * NPAD, F), jnp.float32),
        mesh=_mesh(),
        compiler_params=_sc_params(),
        scratch_types=[
            pltpu.VMEM((2, 2 * KP, 128), jnp.int32),
            pltpu.VMEM((2, 2 * KP, 128), jnp.int32),
            pltpu.VMEM((2, KP, 128, F), jnp.float32),
            pltpu.SemaphoreType.DMA,
            pltpu.SemaphoreType.DMA,
            pltpu.SemaphoreType.DMA,
            pltpu.SemaphoreType.DMA,
            pltpu.SemaphoreType.DMA,
            pltpu.SemaphoreType.DMA,
            pltpu.SemaphoreType.DMA,
            pltpu.SemaphoreType.DMA,
            pltpu.VMEM_SHARED((NPAD, F), jnp.float32),
        ],
    )
    return k(ridx2, cidx2, src, zeros2)


def _prop1_pass(ridx2, cidx2, src1, zeros1):
    k = pl.kernel(
        _make_pass_body(1, KI),
        out_type=jax.ShapeDtypeStruct((NC * NPAD,), jnp.float32),
        mesh=_mesh(),
        compiler_params=_sc_params(),
        scratch_types=[
            pltpu.VMEM((2, 2 * KI, 128), jnp.int32),
            pltpu.VMEM((2, 2 * KI, 128), jnp.int32),
            pltpu.VMEM((2 * KI, 128), jnp.float32),
            pltpu.SemaphoreType.DMA,
            pltpu.SemaphoreType.DMA,
            pltpu.SemaphoreType.DMA,
            pltpu.SemaphoreType.DMA,
            pltpu.SemaphoreType.DMA,
            pltpu.SemaphoreType.DMA,
            pltpu.SemaphoreType.DMA,
            pltpu.SemaphoreType.DMA,
            pltpu.VMEM_SHARED((NPAD,), jnp.float32),
        ],
    )
    return k(ridx2, cidx2, src1, zeros1)


def _tc_dinv_body(deg_ref, dinv_ref):
    d = deg_ref[0, :] + deg_ref[1, :] + 1.0
    dinv_ref[0, :] = lax.rsqrt(d)


def _tc_dinv(deg2):
    return pl.pallas_call(
        _tc_dinv_body,
        out_shape=jax.ShapeDtypeStruct((1, NPAD), jnp.float32),
    )(deg2)


def _tc_xs_body(xT_ref, dinv_ref, xsT_ref):
    xsT_ref[...] = xT_ref[...] * dinv_ref[...]


def _tc_xs(xT, dinv1):
    return pl.pallas_call(
        _tc_xs_body,
        out_shape=jax.ShapeDtypeStruct((F, NPAD), jnp.float32),
    )(xT, dinv1)


def _tc_mid_body(p1aT_ref, p1bT_ref, xsT_ref, dinv_ref, W1T_ref, b1_ref,
                 WfcT_ref, W2T_ref, ssT_ref):
    hp = lax.Precision.HIGHEST
    pre = (p1aT_ref[...] + p1bT_ref[...] + xsT_ref[...]) * dinv_ref[...]
    h = jnp.maximum(
        jnp.dot(W1T_ref[...], pre, preferred_element_type=jnp.float32,
                precision=hp)
        + b1_ref[...], 0.0)
    w2fT = jnp.dot(WfcT_ref[...], W2T_ref[...],
                   preferred_element_type=jnp.float32, precision=hp)
    sT = jnp.dot(w2fT, h, preferred_element_type=jnp.float32,
                 precision=hp)
    ssT_ref[...] = sT * dinv_ref[...]


def _tc_mid(p1aT, p1bT, xsT, dinv1, W1T, b1c, WfcT, W2T):
    return pl.pallas_call(
        _tc_mid_body,
        out_shape=jax.ShapeDtypeStruct((1, NPAD), jnp.float32),
    )(p1aT, p1bT, xsT, dinv1, W1T, b1c, WfcT, W2T)


def _tc_out_body(p2_ref, ss_ref, dinv_ref, b2_ref, Wfc_ref, bfc_ref, out_ref):
    c0 = jnp.sum(b2_ref[0, :] * Wfc_ref[:, 0]) + bfc_ref[0]
    out_ref[0, :] = (p2_ref[0, :] + p2_ref[1, :] + ss_ref[0, :]) * dinv_ref[0, :] + c0


def _tc_out(p2v, ss1, dinv1, b2r, Wfc, bfc):
    return pl.pallas_call(
        _tc_out_body,
        out_shape=jax.ShapeDtypeStruct((1, NPAD), jnp.float32),
    )(p2v, ss1, dinv1, b2r, Wfc, bfc)


@jax.jit
def kernel(x, edge_index, W1, b1, W2, b2, Wfc, bfc):
    r = edge_index[0].astype(jnp.int32)
    c = edge_index[1].astype(jnp.int32)
    npadex = EPAD - N_EDGES
    rpad = (jnp.arange(npadex, dtype=jnp.int32) % N_NODES)
    cpad = N_NODES + (jnp.arange(npadex, dtype=jnp.int32) % (NPAD - N_NODES))
    ridx2 = jnp.concatenate([r, rpad]).reshape(EPAD // 128, 128)
    cidx2 = jnp.concatenate([c, cpad]).reshape(EPAD // 128, 128)

    zeros1 = jnp.zeros((NPAD,), jnp.float32)
    zeros2 = jnp.zeros((NPAD, F), jnp.float32)

    degp = _deg_pass(cidx2, zeros1).reshape(NC, NPAD)
    dinv1 = _tc_dinv(degp)

    xT = jnp.pad(x.T, ((0, F - x.shape[1]), (0, NPAD - N_NODES)))
    xsT = _tc_xs(xT, dinv1)
    xs = xsT.T
    p1p = _prop_pass(ridx2, cidx2, xs, zeros2)
    p1aT, p1bT = p1p[:NPAD].T, p1p[NPAD:].T

    W1T = jnp.pad(W1, ((0, F - W1.shape[0]), (0, 0))).T
    ss1 = _tc_mid(p1aT, p1bT, xsT, dinv1, W1T, b1[:, None], Wfc.T, W2.T)
    ss = ss1.reshape(NPAD)

    p2p = _prop1_pass(ridx2, cidx2, ss, zeros1).reshape(NC, NPAD)
    out1 = _tc_out(p2p, ss1, dinv1, b2[None, :], Wfc, bfc)
    return out1[0, :N_NODES, None]
